# Initial kernel scaffold; baseline (speedup 1.0000x reference)
import jax
import jax.numpy as jnp
from jax.experimental import pallas as pl


def kernel(bbox_nx7, labels):
    raise NotImplementedError("write your pallas kernel here")


# speedup vs baseline: 4.6072x; 4.6072x over previous
import functools

import jax
import jax.numpy as jnp
from jax import lax
from jax.experimental import pallas as pl
from jax.experimental.pallas import tpu as pltpu
from jax.experimental.pallas import tpu_sc as plsc

N_BOXES = 20000
ROWS, LANES = 160, 128
N_PAD = ROWS * LANES
K = 1000
KSEL = 1024
PACKED = 2048
DUMP_SPREAD = 768
MAXOUT = 100
SCORE_THR = 0.01
IOU_THR = 0.5
MIN_SIZE = 0.01
IMG = 256.0

NC, NS = 2, 16
NW = NC * NS
CHUNK = N_PAD // NW
BPW = KSEL // NW


def _select_body(bits_ref, dest_ref):
    bits = bits_ref[...]

    def cnt_ge(t):
        return jnp.sum((bits >= t).astype(jnp.float32))

    def bs_body(_, carry):
        lo, hi = carry
        mid = (lo + hi) // 2
        q = cnt_ge(mid) >= float(K)
        return jnp.where(q, mid, lo), jnp.where(q, hi, mid)

    lo, _ = lax.fori_loop(0, 30, bs_body,
                          (jnp.int32(0), jnp.int32(1 << 30)))
    bstar = lo
    c_hi = jnp.sum((bits > bstar).astype(jnp.float32))
    m = float(K) - c_hi

    il0 = lax.broadcasted_iota(jnp.int32, (LANES, LANES), 0)
    il1 = lax.broadcasted_iota(jnp.int32, (LANES, LANES), 1)
    upper = (il0 < il1).astype(jnp.float32)
    ir0 = lax.broadcasted_iota(jnp.int32, (ROWS, ROWS), 0)
    ir1 = lax.broadcasted_iota(jnp.int32, (ROWS, ROWS), 1)
    lower = (ir1 < ir0).astype(jnp.float32)

    def ex_prefix(mask_f):
        lane_ex = jnp.dot(mask_f, upper,
                          preferred_element_type=jnp.float32)
        rowsum = jnp.sum(mask_f, axis=1, keepdims=True)
        row_ex = jnp.dot(lower, rowsum,
                         preferred_element_type=jnp.float32)
        return lane_ex + row_ex

    tie = bits == bstar
    tie_rank = ex_prefix(tie.astype(jnp.float32))
    sel = (bits > bstar) | (tie & (tie_rank < m))
    pos = ex_prefix(sel.astype(jnp.float32)).astype(jnp.int32)
    lin = (lax.broadcasted_iota(jnp.int32, (ROWS, LANES), 0) * LANES
           + lax.broadcasted_iota(jnp.int32, (ROWS, LANES), 1))
    dump = KSEL + lax.rem(lin, jnp.int32(DUMP_SPREAD))
    dest_ref[...] = jnp.where(sel, pos, dump)


@functools.cache
def _make_compact():
    mesh = plsc.VectorSubcoreMesh(core_axis_name="c", subcore_axis_name="s")
    return functools.partial(
        pl.kernel, mesh=mesh,
        compiler_params=pltpu.CompilerParams(use_tc_tiling_on_sc=False),
        out_type=jax.ShapeDtypeStruct((PACKED,), jnp.int32),
        scratch_types=[
            pltpu.VMEM((CHUNK,), jnp.int32),
            pltpu.VMEM((CHUNK,), jnp.int32),
            pltpu.VMEM((32,), jnp.int32),
            pltpu.VMEM((32,), jnp.int32),
            pltpu.SemaphoreType.DMA,
        ],
    )(_compact_body)


def _compact_body(dest_hbm, packed_hbm, pos_v, src_v, zpos_v, zval_v, sem):
    wid = lax.axis_index("s") * NC + lax.axis_index("c")
    base = wid * CHUNK
    pltpu.sync_copy(dest_hbm.at[pl.ds(base, CHUNK)], pos_v)
    for j in range(CHUNK // 16):
        src_v[pl.ds(j * 16, 16)] = lax.iota(jnp.int32, 16) + (base + j * 16)
    pltpu.async_copy(src_v, packed_hbm.at[pos_v], sem).wait()

    @pl.when(wid == 0)
    def _():
        zpos_v[pl.ds(0, 16)] = lax.iota(jnp.int32, 16) + K
        zpos_v[pl.ds(16, 16)] = lax.iota(jnp.int32, 16) + (K + 16)
        zval_v[pl.ds(0, 16)] = jnp.zeros((16,), jnp.int32)
        zval_v[pl.ds(16, 16)] = jnp.zeros((16,), jnp.int32)
        pltpu.async_copy(zval_v, packed_hbm.at[zpos_v], sem).wait()


@functools.cache
def _make_gather():
    mesh = plsc.VectorSubcoreMesh(core_axis_name="c", subcore_axis_name="s")
    return functools.partial(
        pl.kernel, mesh=mesh,
        compiler_params=pltpu.CompilerParams(use_tc_tiling_on_sc=False),
        out_type=(jax.ShapeDtypeStruct((KSEL, 8), jnp.float32),
                  jax.ShapeDtypeStruct((KSEL,), jnp.int32)),
        scratch_types=[
            pltpu.VMEM((BPW,), jnp.int32),
            pltpu.VMEM((BPW, 8), jnp.float32),
            pltpu.VMEM((BPW,), jnp.int32),
            pltpu.SemaphoreType.DMA,
            pltpu.SemaphoreType.DMA,
        ],
    )(_gather_body)


def _gather_body(packed_hbm, bbox_hbm, lab_hbm, boxes_out, labs_out,
                 idx_v, rows_v, lab_v, sem, sem2):
    wid = lax.axis_index("s") * NC + lax.axis_index("c")
    base = wid * BPW
    pltpu.sync_copy(packed_hbm.at[pl.ds(base, BPW)], idx_v)
    cp_rows = pltpu.async_copy(bbox_hbm.at[idx_v], rows_v, sem)
    cp_lab = pltpu.async_copy(lab_hbm.at[idx_v], lab_v, sem2)
    cp_rows.wait()
    cp_lab.wait()
    pltpu.sync_copy(rows_v, boxes_out.at[pl.ds(base, BPW)])
    pltpu.sync_copy(lab_v, labs_out.at[pl.ds(base, BPW)])


def _nms_body(x0_ref, y0_ref, z0_ref, x1_ref, y1_ref, z1_ref, sc_ref,
              lab_ref, dets_ref, labout_ref):
    sc = sc_ref[...]
    labf = lab_ref[...].astype(jnp.float32)
    cx0 = jnp.clip(x0_ref[...], 0.0, IMG)
    cy0 = jnp.clip(y0_ref[...], 0.0, IMG)
    cz0 = jnp.clip(z0_ref[...], 0.0, IMG)
    cx1 = jnp.clip(x1_ref[...], 0.0, IMG)
    cy1 = jnp.clip(y1_ref[...], 0.0, IMG)
    cz1 = jnp.clip(z1_ref[...], 0.0, IMG)

    slot = (lax.broadcasted_iota(jnp.int32, (8, LANES), 0) * LANES
            + lax.broadcasted_iota(jnp.int32, (8, LANES), 1))
    real = slot < K
    valid = ((sc > SCORE_THR)
             & (cx1 - cx0 > MIN_SIZE)
             & (cy1 - cy0 > MIN_SIZE)
             & (cz1 - cz0 > MIN_SIZE)
             & real)

    zero = jnp.float32(0.0)
    mc = jnp.maximum(
        jnp.maximum(jnp.max(jnp.where(real, cx0, zero)),
                    jnp.max(jnp.where(real, cy0, zero))),
        jnp.maximum(
            jnp.maximum(jnp.max(jnp.where(real, cz0, zero)),
                        jnp.max(jnp.where(real, cx1, zero))),
            jnp.maximum(jnp.max(jnp.where(real, cy1, zero)),
                        jnp.max(jnp.where(real, cz1, zero)))))
    off = labf * (mc + 1.0)
    bx0 = cx0 + off
    by0 = cy0 + off
    bz0 = cz0 + off
    bx1 = cx1 + off
    by1 = cy1 + off
    bz1 = cz1 + off
    vol = (jnp.maximum(bx1 - bx0, 0.0)
           * jnp.maximum(by1 - by0, 0.0)
           * jnp.maximum(bz1 - bz0, 0.0))

    dets_ref[...] = jnp.zeros((MAXOUT, LANES), jnp.float32)
    labout_ref[...] = jnp.zeros((MAXOUT, LANES), jnp.int32)

    lane = lax.broadcasted_iota(jnp.int32, (1, LANES), 1)
    neg_inf = jnp.float32(-jnp.inf)

    def body(_, carry):
        avail_f, npick = carry
        avail = avail_f > zero
        cur = jnp.where(avail, sc, neg_inf)
        mval = jnp.max(cur)
        has = mval > neg_inf
        pick = jnp.min(jnp.where(cur == mval, slot, jnp.int32(1 << 30)))
        pick = jnp.where(has, pick, 0)
        pmask = (slot == pick).astype(jnp.float32)

        def ext(plane):
            return jnp.sum(plane * pmask)

        px0, py0, pz0 = ext(bx0), ext(by0), ext(bz0)
        px1, py1, pz1 = ext(bx1), ext(by1), ext(bz1)
        pvol = ext(vol)
        ltx = jnp.maximum(px0, bx0)
        lty = jnp.maximum(py0, by0)
        ltz = jnp.maximum(pz0, bz0)
        rbx = jnp.minimum(px1, bx1)
        rby = jnp.minimum(py1, by1)
        rbz = jnp.minimum(pz1, bz1)
        inter = (jnp.maximum(rbx - ltx, 0.0)
                 * jnp.maximum(rby - lty, 0.0)
                 * jnp.maximum(rbz - ltz, 0.0))
        union = pvol + vol - inter
        iou = inter / jnp.maximum(union, 1e-8)
        sup = (iou > IOU_THR) | (slot == pick)
        new_avail = avail & jnp.logical_not(sup & has)

        do_out = has & (npick < MAXOUT)

        @pl.when(do_out)
        def _():
            psc = ext(sc)
            row = (jnp.where(lane == 0, ext(cx0), zero)
                   + jnp.where(lane == 1, ext(cy0), zero)
                   + jnp.where(lane == 2, ext(cz0), zero)
                   + jnp.where(lane == 3, ext(cx1), zero)
                   + jnp.where(lane == 4, ext(cy1), zero)
                   + jnp.where(lane == 5, ext(cz1), zero)
                   + jnp.where((lane == 6) | (lane == 7), psc, zero))
            dets_ref[pl.ds(npick, 1), :] = row
            plab = ext(labf).astype(jnp.int32)
            labout_ref[pl.ds(npick, 1), :] = jnp.broadcast_to(
                plab, (1, LANES))

        npick = npick + jnp.where(has, jnp.int32(1), jnp.int32(0))
        return new_avail.astype(jnp.float32), npick

    lax.fori_loop(0, K, body,
                  (valid.astype(jnp.float32), jnp.int32(0)))


def _run_select(bits2d):
    return pl.pallas_call(
        _select_body,
        out_shape=jax.ShapeDtypeStruct((ROWS, LANES), jnp.int32),
    )(bits2d)


def _run_nms(planes, labp):
    return pl.pallas_call(
        _nms_body,
        out_shape=(jax.ShapeDtypeStruct((MAXOUT, LANES), jnp.float32),
                   jax.ShapeDtypeStruct((MAXOUT, LANES), jnp.int32)),
    )(*planes, labp)


def kernel(bbox_nx7, labels):
    scores = bbox_nx7[:, 6]
    bits = lax.bitcast_convert_type(scores, jnp.int32)
    bits2d = jnp.pad(bits, (0, N_PAD - N_BOXES),
                     constant_values=-1).reshape(ROWS, LANES)
    dest = _run_select(bits2d).reshape(N_PAD)
    packed = _make_compact()(dest)
    bbox_pad = jnp.pad(bbox_nx7, ((0, N_PAD - N_BOXES), (0, 1)))
    lab_pad = jnp.pad(labels, (0, N_PAD - N_BOXES))
    boxes_sel, labs_sel = _make_gather()(packed, bbox_pad, lab_pad)
    planes = [boxes_sel[:, i].reshape(8, LANES) for i in range(7)]
    labp = labs_sel.reshape(8, LANES)
    dets, labout = _run_nms(planes, labp)
    return dets[:, :8], labout[:, 0]


# speedup vs baseline: 8.0578x; 1.7490x over previous
import functools

import jax
import jax.numpy as jnp
from jax import lax
from jax.experimental import pallas as pl
from jax.experimental.pallas import tpu as pltpu
from jax.experimental.pallas import tpu_sc as plsc

N_BOXES = 20000
ROWS, LANES = 160, 128
N_PAD = ROWS * LANES
K = 1000
KSEL = 1024
PACKED = KSEL + 32 + N_PAD
MAXOUT = 100
SCORE_THR = 0.01
IOU_THR = 0.5
MIN_SIZE = 0.01
IMG = 256.0

NC, NS = 2, 16
NW = NC * NS
CHUNK = N_PAD // NW
BPW = KSEL // NW


def _select_body(bits_ref, dest_ref):
    bits = bits_ref[...]

    def cnt_ge(t):
        return jnp.sum((bits >= t).astype(jnp.float32))

    def bs_body(_, carry):
        lo, hi = carry
        mid = (lo + hi) // 2
        q = cnt_ge(mid) >= float(K)
        return jnp.where(q, mid, lo), jnp.where(q, hi, mid)

    lo, _ = lax.fori_loop(0, 30, bs_body,
                          (jnp.int32(0), jnp.int32(1 << 30)))
    bstar = lo
    c_hi = jnp.sum((bits > bstar).astype(jnp.float32))
    m = float(K) - c_hi

    il0 = lax.broadcasted_iota(jnp.int32, (LANES, LANES), 0)
    il1 = lax.broadcasted_iota(jnp.int32, (LANES, LANES), 1)
    upper = (il0 < il1).astype(jnp.float32)
    ir0 = lax.broadcasted_iota(jnp.int32, (ROWS, ROWS), 0)
    ir1 = lax.broadcasted_iota(jnp.int32, (ROWS, ROWS), 1)
    lower = (ir1 < ir0).astype(jnp.float32)

    def ex_prefix(mask_f):
        lane_ex = jnp.dot(mask_f, upper,
                          preferred_element_type=jnp.float32)
        rowsum = jnp.sum(mask_f, axis=1, keepdims=True)
        row_ex = jnp.dot(lower, rowsum,
                         preferred_element_type=jnp.float32)
        return lane_ex + row_ex

    tie = bits == bstar
    tie_rank = ex_prefix(tie.astype(jnp.float32))
    sel = (bits > bstar) | (tie & (tie_rank < m))
    pos = ex_prefix(sel.astype(jnp.float32)).astype(jnp.int32)
    lin = (lax.broadcasted_iota(jnp.int32, (ROWS, LANES), 0) * LANES
           + lax.broadcasted_iota(jnp.int32, (ROWS, LANES), 1))
    dump = KSEL + 32 + lin
    dest_ref[...] = jnp.where(sel, pos, dump)


@functools.cache
def _make_compact():
    mesh = plsc.VectorSubcoreMesh(core_axis_name="c", subcore_axis_name="s")
    return functools.partial(
        pl.kernel, mesh=mesh,
        compiler_params=pltpu.CompilerParams(use_tc_tiling_on_sc=False),
        out_type=jax.ShapeDtypeStruct((PACKED,), jnp.int32),
        scratch_types=[
            pltpu.VMEM((CHUNK,), jnp.int32),
            pltpu.VMEM((CHUNK,), jnp.int32),
            pltpu.VMEM((32,), jnp.int32),
            pltpu.VMEM((32,), jnp.int32),
            pltpu.SemaphoreType.DMA,
        ],
    )(_compact_body)


def _compact_body(dest_hbm, packed_hbm, pos_v, src_v, zpos_v, zval_v, sem):
    wid = lax.axis_index("s") * NC + lax.axis_index("c")
    base = wid * CHUNK
    pltpu.sync_copy(dest_hbm.at[pl.ds(base, CHUNK)], pos_v)
    for j in range(CHUNK // 16):
        src_v[pl.ds(j * 16, 16)] = lax.iota(jnp.int32, 16) + (base + j * 16)
    pltpu.async_copy(src_v, packed_hbm.at[pos_v], sem).wait()

    @pl.when(wid == 0)
    def _():
        zpos_v[pl.ds(0, 16)] = lax.iota(jnp.int32, 16) + K
        zpos_v[pl.ds(16, 16)] = lax.iota(jnp.int32, 16) + (K + 16)
        zval_v[pl.ds(0, 16)] = jnp.zeros((16,), jnp.int32)
        zval_v[pl.ds(16, 16)] = jnp.zeros((16,), jnp.int32)
        pltpu.async_copy(zval_v, packed_hbm.at[zpos_v], sem).wait()


@functools.cache
def _make_gather():
    mesh = plsc.VectorSubcoreMesh(core_axis_name="c", subcore_axis_name="s")
    return functools.partial(
        pl.kernel, mesh=mesh,
        compiler_params=pltpu.CompilerParams(use_tc_tiling_on_sc=False),
        out_type=(jax.ShapeDtypeStruct((KSEL, 8), jnp.float32),
                  jax.ShapeDtypeStruct((KSEL,), jnp.int32)),
        scratch_types=[
            pltpu.VMEM((BPW,), jnp.int32),
            pltpu.VMEM((BPW, 8), jnp.float32),
            pltpu.VMEM((BPW,), jnp.int32),
            pltpu.SemaphoreType.DMA,
            pltpu.SemaphoreType.DMA,
        ],
    )(_gather_body)


def _gather_body(packed_hbm, bbox_hbm, lab_hbm, boxes_out, labs_out,
                 idx_v, rows_v, lab_v, sem, sem2):
    wid = lax.axis_index("s") * NC + lax.axis_index("c")
    base = wid * BPW
    pltpu.sync_copy(packed_hbm.at[pl.ds(base, BPW)], idx_v)
    cp_rows = pltpu.async_copy(bbox_hbm.at[idx_v], rows_v, sem)
    cp_lab = pltpu.async_copy(lab_hbm.at[idx_v], lab_v, sem2)
    cp_rows.wait()
    cp_lab.wait()
    pltpu.sync_copy(rows_v, boxes_out.at[pl.ds(base, BPW)])
    pltpu.sync_copy(lab_v, labs_out.at[pl.ds(base, BPW)])


def _nms_body(x0_ref, y0_ref, z0_ref, x1_ref, y1_ref, z1_ref, sc_ref,
              lab_ref, dets_ref, labout_ref):
    sc = sc_ref[...]
    labf = lab_ref[...].astype(jnp.float32)
    cx0 = jnp.clip(x0_ref[...], 0.0, IMG)
    cy0 = jnp.clip(y0_ref[...], 0.0, IMG)
    cz0 = jnp.clip(z0_ref[...], 0.0, IMG)
    cx1 = jnp.clip(x1_ref[...], 0.0, IMG)
    cy1 = jnp.clip(y1_ref[...], 0.0, IMG)
    cz1 = jnp.clip(z1_ref[...], 0.0, IMG)

    slot = (lax.broadcasted_iota(jnp.int32, (8, LANES), 0) * LANES
            + lax.broadcasted_iota(jnp.int32, (8, LANES), 1))
    real = slot < K
    valid = ((sc > SCORE_THR)
             & (cx1 - cx0 > MIN_SIZE)
             & (cy1 - cy0 > MIN_SIZE)
             & (cz1 - cz0 > MIN_SIZE)
             & real)

    zero = jnp.float32(0.0)
    mc = jnp.maximum(
        jnp.maximum(jnp.max(jnp.where(real, cx0, zero)),
                    jnp.max(jnp.where(real, cy0, zero))),
        jnp.maximum(
            jnp.maximum(jnp.max(jnp.where(real, cz0, zero)),
                        jnp.max(jnp.where(real, cx1, zero))),
            jnp.maximum(jnp.max(jnp.where(real, cy1, zero)),
                        jnp.max(jnp.where(real, cz1, zero)))))
    off = labf * (mc + 1.0)
    bx0 = cx0 + off
    by0 = cy0 + off
    bz0 = cz0 + off
    bx1 = cx1 + off
    by1 = cy1 + off
    bz1 = cz1 + off
    vol = (jnp.maximum(bx1 - bx0, 0.0)
           * jnp.maximum(by1 - by0, 0.0)
           * jnp.maximum(bz1 - bz0, 0.0))

    dets_ref[...] = jnp.zeros((MAXOUT, LANES), jnp.float32)
    labout_ref[...] = jnp.zeros((MAXOUT, LANES), jnp.int32)

    lane = lax.broadcasted_iota(jnp.int32, (1, LANES), 1)
    neg_inf = jnp.float32(-jnp.inf)

    def body(_, carry):
        avail_f, npick = carry
        avail = avail_f > zero
        cur = jnp.where(avail, sc, neg_inf)
        mval = jnp.max(cur)
        has = mval > neg_inf
        pick = jnp.min(jnp.where(cur == mval, slot, jnp.int32(1 << 30)))
        pick = jnp.where(has, pick, 0)
        pmask = (slot == pick).astype(jnp.float32)

        def ext(plane):
            return jnp.sum(plane * pmask)

        px0, py0, pz0 = ext(bx0), ext(by0), ext(bz0)
        px1, py1, pz1 = ext(bx1), ext(by1), ext(bz1)
        pvol = ext(vol)
        ltx = jnp.maximum(px0, bx0)
        lty = jnp.maximum(py0, by0)
        ltz = jnp.maximum(pz0, bz0)
        rbx = jnp.minimum(px1, bx1)
        rby = jnp.minimum(py1, by1)
        rbz = jnp.minimum(pz1, bz1)
        inter = (jnp.maximum(rbx - ltx, 0.0)
                 * jnp.maximum(rby - lty, 0.0)
                 * jnp.maximum(rbz - ltz, 0.0))
        union = pvol + vol - inter
        iou = inter / jnp.maximum(union, 1e-8)
        sup = (iou > IOU_THR) | (slot == pick)
        new_avail = avail & jnp.logical_not(sup & has)

        do_out = has & (npick < MAXOUT)

        @pl.when(do_out)
        def _():
            psc = ext(sc)
            row = (jnp.where(lane == 0, ext(cx0), zero)
                   + jnp.where(lane == 1, ext(cy0), zero)
                   + jnp.where(lane == 2, ext(cz0), zero)
                   + jnp.where(lane == 3, ext(cx1), zero)
                   + jnp.where(lane == 4, ext(cy1), zero)
                   + jnp.where(lane == 5, ext(cz1), zero)
                   + jnp.where((lane == 6) | (lane == 7), psc, zero))
            dets_ref[pl.ds(npick, 1), :] = row
            plab = ext(labf).astype(jnp.int32)
            labout_ref[pl.ds(npick, 1), :] = jnp.broadcast_to(
                plab, (1, LANES))

        npick = npick + jnp.where(has, jnp.int32(1), jnp.int32(0))
        return new_avail.astype(jnp.float32), npick

    lax.fori_loop(0, K, body,
                  (valid.astype(jnp.float32), jnp.int32(0)))


def _run_select(bits2d):
    return pl.pallas_call(
        _select_body,
        out_shape=jax.ShapeDtypeStruct((ROWS, LANES), jnp.int32),
    )(bits2d)


def _run_nms(planes, labp):
    return pl.pallas_call(
        _nms_body,
        out_shape=(jax.ShapeDtypeStruct((MAXOUT, LANES), jnp.float32),
                   jax.ShapeDtypeStruct((MAXOUT, LANES), jnp.int32)),
    )(*planes, labp)


def kernel(bbox_nx7, labels):
    scores = bbox_nx7[:, 6]
    bits = lax.bitcast_convert_type(scores, jnp.int32)
    bits2d = jnp.pad(bits, (0, N_PAD - N_BOXES),
                     constant_values=-1).reshape(ROWS, LANES)
    dest = _run_select(bits2d).reshape(N_PAD)
    packed = _make_compact()(dest)
    bbox_pad = jnp.pad(bbox_nx7, ((0, N_PAD - N_BOXES), (0, 1)))
    lab_pad = jnp.pad(labels, (0, N_PAD - N_BOXES))
    boxes_sel, labs_sel = _make_gather()(packed, bbox_pad, lab_pad)
    planes = [boxes_sel[:, i].reshape(8, LANES) for i in range(7)]
    labp = labs_sel.reshape(8, LANES)
    dets, labout = _run_nms(planes, labp)
    return dets[:, :8], labout[:, 0]


# speedup vs baseline: 26.8758x; 3.3354x over previous
import functools

import jax
import jax.numpy as jnp
from jax import lax
from jax.experimental import pallas as pl
from jax.experimental.pallas import tpu as pltpu
from jax.experimental.pallas import tpu_sc as plsc

N_BOXES = 20000
ROWS, LANES = 160, 128
N_PAD = ROWS * LANES
K = 1000
KSEL = 1024
PACKED = KSEL + 32 + N_PAD
MAXOUT = 100
SCORE_THR = 0.01
IOU_THR = 0.5
MIN_SIZE = 0.01
IMG = 256.0

NC, NS = 2, 16
NW = NC * NS
CHUNK = N_PAD // NW
BPW = KSEL // NW


def _select_body(bits_ref, dest_ref):
    bits = bits_ref[...]

    def cnt_ge(t):
        return jnp.sum((bits >= t).astype(jnp.float32))

    def bs_body(_, carry):
        lo, hi = carry
        mid = (lo + hi) // 2
        q = cnt_ge(mid) >= float(K)
        return jnp.where(q, mid, lo), jnp.where(q, hi, mid)

    lo, _ = lax.fori_loop(0, 30, bs_body,
                          (jnp.int32(0), jnp.int32(1 << 30)))
    bstar = lo
    c_hi = jnp.sum((bits > bstar).astype(jnp.float32))
    m = float(K) - c_hi

    il0 = lax.broadcasted_iota(jnp.int32, (LANES, LANES), 0)
    il1 = lax.broadcasted_iota(jnp.int32, (LANES, LANES), 1)
    upper = (il0 < il1).astype(jnp.float32)
    ir0 = lax.broadcasted_iota(jnp.int32, (ROWS, ROWS), 0)
    ir1 = lax.broadcasted_iota(jnp.int32, (ROWS, ROWS), 1)
    lower = (ir1 < ir0).astype(jnp.float32)

    def ex_prefix(mask_f):
        lane_ex = jnp.dot(mask_f, upper,
                          preferred_element_type=jnp.float32)
        rowsum = jnp.sum(mask_f, axis=1, keepdims=True)
        row_ex = jnp.dot(lower, rowsum,
                         preferred_element_type=jnp.float32)
        return lane_ex + row_ex

    tie = bits == bstar
    tie_rank = ex_prefix(tie.astype(jnp.float32))
    sel = (bits > bstar) | (tie & (tie_rank < m))
    pos = ex_prefix(sel.astype(jnp.float32)).astype(jnp.int32)
    lin = (lax.broadcasted_iota(jnp.int32, (ROWS, LANES), 0) * LANES
           + lax.broadcasted_iota(jnp.int32, (ROWS, LANES), 1))
    dump = KSEL + 32 + lin
    dest_ref[...] = jnp.where(sel, pos, dump)


@functools.cache
def _make_compact():
    mesh = plsc.VectorSubcoreMesh(core_axis_name="c", subcore_axis_name="s")
    return functools.partial(
        pl.kernel, mesh=mesh,
        compiler_params=pltpu.CompilerParams(use_tc_tiling_on_sc=False),
        out_type=jax.ShapeDtypeStruct((PACKED,), jnp.int32),
        scratch_types=[
            pltpu.VMEM((CHUNK,), jnp.int32),
            pltpu.VMEM((CHUNK,), jnp.int32),
            pltpu.VMEM((32,), jnp.int32),
            pltpu.VMEM((32,), jnp.int32),
            pltpu.SemaphoreType.DMA,
        ],
    )(_compact_body)


def _compact_body(dest_hbm, packed_hbm, pos_v, src_v, zpos_v, zval_v, sem):
    wid = lax.axis_index("s") * NC + lax.axis_index("c")
    base = wid * CHUNK
    pltpu.sync_copy(dest_hbm.at[pl.ds(base, CHUNK)], pos_v)
    for j in range(CHUNK // 16):
        src_v[pl.ds(j * 16, 16)] = lax.iota(jnp.int32, 16) + (base + j * 16)
    pltpu.async_copy(src_v, packed_hbm.at[pos_v], sem).wait()

    @pl.when(wid == 0)
    def _():
        zpos_v[pl.ds(0, 16)] = lax.iota(jnp.int32, 16) + K
        zpos_v[pl.ds(16, 16)] = lax.iota(jnp.int32, 16) + (K + 16)
        zval_v[pl.ds(0, 16)] = jnp.zeros((16,), jnp.int32)
        zval_v[pl.ds(16, 16)] = jnp.zeros((16,), jnp.int32)
        pltpu.async_copy(zval_v, packed_hbm.at[zpos_v], sem).wait()


@functools.cache
def _make_gather():
    mesh = plsc.VectorSubcoreMesh(core_axis_name="c", subcore_axis_name="s")
    return functools.partial(
        pl.kernel, mesh=mesh,
        compiler_params=pltpu.CompilerParams(use_tc_tiling_on_sc=False),
        out_type=(jax.ShapeDtypeStruct((KSEL, 8), jnp.float32),
                  jax.ShapeDtypeStruct((KSEL,), jnp.int32)),
        scratch_types=[
            pltpu.VMEM((BPW,), jnp.int32),
            pltpu.VMEM((BPW, 8), jnp.float32),
            pltpu.VMEM((BPW,), jnp.int32),
            pltpu.SemaphoreType.DMA,
            pltpu.SemaphoreType.DMA,
        ],
    )(_gather_body)


def _gather_body(packed_hbm, bbox_hbm, lab_hbm, boxes_out, labs_out,
                 idx_v, rows_v, lab_v, sem, sem2):
    wid = lax.axis_index("s") * NC + lax.axis_index("c")
    base = wid * BPW
    pltpu.sync_copy(packed_hbm.at[pl.ds(base, BPW)], idx_v)
    cp_rows = pltpu.async_copy(bbox_hbm.at[idx_v], rows_v, sem)
    cp_lab = pltpu.async_copy(lab_hbm.at[idx_v], lab_v, sem2)
    cp_rows.wait()
    cp_lab.wait()
    pltpu.sync_copy(rows_v, boxes_out.at[pl.ds(base, BPW)])
    pltpu.sync_copy(lab_v, labs_out.at[pl.ds(base, BPW)])


def _nms_body(x0_ref, y0_ref, z0_ref, x1_ref, y1_ref, z1_ref, sc_ref,
              lab_ref, dets_ref, labout_ref):
    sc = sc_ref[...]
    labf = lab_ref[...].astype(jnp.float32)
    cx0 = jnp.clip(x0_ref[...], 0.0, IMG)
    cy0 = jnp.clip(y0_ref[...], 0.0, IMG)
    cz0 = jnp.clip(z0_ref[...], 0.0, IMG)
    cx1 = jnp.clip(x1_ref[...], 0.0, IMG)
    cy1 = jnp.clip(y1_ref[...], 0.0, IMG)
    cz1 = jnp.clip(z1_ref[...], 0.0, IMG)

    slot = (lax.broadcasted_iota(jnp.int32, (8, LANES), 0) * LANES
            + lax.broadcasted_iota(jnp.int32, (8, LANES), 1))
    real = slot < K
    valid = ((sc > SCORE_THR)
             & (cx1 - cx0 > MIN_SIZE)
             & (cy1 - cy0 > MIN_SIZE)
             & (cz1 - cz0 > MIN_SIZE)
             & real)

    zero = jnp.float32(0.0)
    mc = jnp.maximum(
        jnp.maximum(jnp.max(jnp.where(real, cx0, zero)),
                    jnp.max(jnp.where(real, cy0, zero))),
        jnp.maximum(
            jnp.maximum(jnp.max(jnp.where(real, cz0, zero)),
                        jnp.max(jnp.where(real, cx1, zero))),
            jnp.maximum(jnp.max(jnp.where(real, cy1, zero)),
                        jnp.max(jnp.where(real, cz1, zero)))))
    off = labf * (mc + 1.0)
    bx0 = cx0 + off
    by0 = cy0 + off
    bz0 = cz0 + off
    bx1 = cx1 + off
    by1 = cy1 + off
    bz1 = cz1 + off
    vol = (jnp.maximum(bx1 - bx0, 0.0)
           * jnp.maximum(by1 - by0, 0.0)
           * jnp.maximum(bz1 - bz0, 0.0))

    dets_ref[...] = jnp.zeros((MAXOUT, LANES), jnp.float32)
    labout_ref[...] = jnp.zeros((MAXOUT, LANES), jnp.int32)

    lane = lax.broadcasted_iota(jnp.int32, (1, LANES), 1)
    neg_inf = jnp.float32(-jnp.inf)

    def cond(carry):
        _, _, done, _ = carry
        return done == 0

    def body(carry):
        avail_f, npick, _, it = carry
        avail = avail_f > zero
        cur = jnp.where(avail, sc, neg_inf)
        mval = jnp.max(cur)
        has = mval > neg_inf
        pick = jnp.min(jnp.where(cur == mval, slot, jnp.int32(1 << 30)))
        pick = jnp.where(has, pick, 0)
        pmask = (slot == pick).astype(jnp.float32)

        def ext(plane):
            return jnp.sum(plane * pmask)

        px0, py0, pz0 = ext(bx0), ext(by0), ext(bz0)
        px1, py1, pz1 = ext(bx1), ext(by1), ext(bz1)
        pvol = ext(vol)
        ltx = jnp.maximum(px0, bx0)
        lty = jnp.maximum(py0, by0)
        ltz = jnp.maximum(pz0, bz0)
        rbx = jnp.minimum(px1, bx1)
        rby = jnp.minimum(py1, by1)
        rbz = jnp.minimum(pz1, bz1)
        inter = (jnp.maximum(rbx - ltx, 0.0)
                 * jnp.maximum(rby - lty, 0.0)
                 * jnp.maximum(rbz - ltz, 0.0))
        union = pvol + vol - inter
        iou = inter / jnp.maximum(union, 1e-8)
        sup = (iou > IOU_THR) | (slot == pick)
        new_avail = avail & jnp.logical_not(sup & has)

        do_out = has & (npick < MAXOUT)

        @pl.when(do_out)
        def _():
            psc = ext(sc)
            row = (jnp.where(lane == 0, ext(cx0), zero)
                   + jnp.where(lane == 1, ext(cy0), zero)
                   + jnp.where(lane == 2, ext(cz0), zero)
                   + jnp.where(lane == 3, ext(cx1), zero)
                   + jnp.where(lane == 4, ext(cy1), zero)
                   + jnp.where(lane == 5, ext(cz1), zero)
                   + jnp.where((lane == 6) | (lane == 7), psc, zero))
            dets_ref[pl.ds(npick, 1), :] = row
            plab = ext(labf).astype(jnp.int32)
            labout_ref[pl.ds(npick, 1), :] = jnp.broadcast_to(
                plab, (1, LANES))

        npick = npick + jnp.where(has, jnp.int32(1), jnp.int32(0))
        done = jnp.where(
            jnp.logical_not(has) | (npick >= MAXOUT) | (it + 1 >= K),
            jnp.int32(1), jnp.int32(0))
        return new_avail.astype(jnp.float32), npick, done, it + 1

    lax.while_loop(cond, body,
                   (valid.astype(jnp.float32), jnp.int32(0),
                    jnp.int32(0), jnp.int32(0)))


def _run_select(bits2d):
    return pl.pallas_call(
        _select_body,
        out_shape=jax.ShapeDtypeStruct((ROWS, LANES), jnp.int32),
    )(bits2d)


def _run_nms(planes, labp):
    return pl.pallas_call(
        _nms_body,
        out_shape=(jax.ShapeDtypeStruct((MAXOUT, LANES), jnp.float32),
                   jax.ShapeDtypeStruct((MAXOUT, LANES), jnp.int32)),
    )(*planes, labp)


def kernel(bbox_nx7, labels):
    scores = bbox_nx7[:, 6]
    bits = lax.bitcast_convert_type(scores, jnp.int32)
    bits2d = jnp.pad(bits, (0, N_PAD - N_BOXES),
                     constant_values=-1).reshape(ROWS, LANES)
    dest = _run_select(bits2d).reshape(N_PAD)
    packed = _make_compact()(dest)
    bbox_pad = jnp.pad(bbox_nx7, ((0, N_PAD - N_BOXES), (0, 1)))
    lab_pad = jnp.pad(labels, (0, N_PAD - N_BOXES))
    boxes_sel, labs_sel = _make_gather()(packed, bbox_pad, lab_pad)
    planes = [boxes_sel[:, i].reshape(8, LANES) for i in range(7)]
    labp = labs_sel.reshape(8, LANES)
    dets, labout = _run_nms(planes, labp)
    return dets[:, :8], labout[:, 0]
